# baseline (speedup 1.0000x reference)
import functools

import jax
import jax.numpy as jnp
from jax import lax
from jax.experimental import pallas as pl
from jax.experimental.pallas import tpu as pltpu
from jax.experimental.pallas import tpu_sc as plsc

_N = 10000
_NP = 10240
_E = 320000
_DIN = 128
_H = 64
_B = 8
_EPS = 1e-5
_SCALE = float((1.0 + _EPS) ** -0.5)

_NC = 2
_NS = 16
_NT = _NC * _NS
_EP = _E // _NT
_CH = 10000
_NCH = _E // _CH


def _sc_mesh():
    return plsc.VectorSubcoreMesh(
        core_axis_name="c", subcore_axis_name="s",
        num_cores=_NC, num_subcores=_NS)


_SC_PARAMS = pltpu.CompilerParams(needs_layout_passes=False)


def _wid():
    return lax.axis_index("s") * _NC + lax.axis_index("c")


def _deg_body(dst_hbm, w_hbm, out_hbm, dst_buf, w_buf, deg_loc):
    w = _wid()
    base = pl.multiple_of(w * _EP, 8)
    pltpu.sync_copy(dst_hbm.at[pl.ds(base, _EP)], dst_buf)
    pltpu.sync_copy(w_hbm.at[pl.ds(base, _EP)], w_buf)

    def _zero(i, carry):
        deg_loc[pl.ds(i * 16, 16)] = jnp.zeros((16,), jnp.float32)
        return carry
    lax.fori_loop(0, _NP // 16, _zero, 0)

    def _edge(i, carry):
        d = dst_buf[pl.ds(i * 16, 16)]
        ww = jnp.abs(w_buf[pl.ds(i * 16, 16)])
        plsc.addupdate_scatter(deg_loc, [d], ww)
        return carry
    lax.fori_loop(0, _EP // 16, _edge, 0)

    pltpu.sync_copy(deg_loc, out_hbm.at[w])


def _deg_partials(dst, ew):
    return pl.kernel(
        _deg_body,
        out_type=jax.ShapeDtypeStruct((_NT, _NP), jnp.float32),
        mesh=_sc_mesh(),
        scratch_types=[
            pltpu.VMEM((_EP,), jnp.int32),
            pltpu.VMEM((_EP,), jnp.float32),
            pltpu.VMEM((_NP,), jnp.float32),
        ],
        compiler_params=_SC_PARAMS,
    )(dst, ew)


_DT = 4
_EH = _E // 2
_ECH = 3200
_NECH = _EH // _ECH
_GRP = _ECH // 16
_UNR = 4


def _scat_body(p_hbm, e3_hbm, out0_hbm, out1_hbm,
               p_rows, acc_rows, ebuf, sem0, sem1):
    w = _wid()
    half = w & 1
    dgrp = w >> 1
    r0 = pl.multiple_of(dgrp * _DT, 4)
    pltpu.sync_copy(p_hbm.at[pl.ds(r0, _DT)], p_rows)

    sems = [sem0, sem1]

    def _edge_cp(c, b):
        base = pl.multiple_of(half * _EH + c * _ECH, 128)
        return pltpu.make_async_copy(
            e3_hbm.at[:, pl.ds(base, _ECH)], ebuf.at[b], sems[b])

    _edge_cp(0, 0).start()
    _edge_cp(1, 1).start()

    keep = jnp.where(half == 0, 1.0, 0.0)

    def _init(i, carry):
        for j in range(_DT):
            acc_rows[j, pl.ds(i * 16, 16)] = p_rows[j, pl.ds(i * 16, 16)] * keep
        return carry
    lax.fori_loop(0, _NP // 16, _init, 0)

    def _pair(k, carry):
        for b in range(2):
            c = k * 2 + b
            _edge_cp(c, b).wait()

            def _group(g, carry2):
                for u in range(_UNR):
                    o = g * (16 * _UNR) + u * 16
                    s = ebuf[b, 0, pl.ds(o, 16)]
                    d = ebuf[b, 1, pl.ds(o, 16)]
                    ww = jnp.abs(plsc.bitcast(ebuf[b, 2, pl.ds(o, 16)],
                                              jnp.float32))
                    for j in range(_DT):
                        rj = jnp.full((16,), j, jnp.int32)
                        vals = plsc.load_gather(p_rows, [rj, s])
                        plsc.addupdate_scatter(acc_rows, [rj, d], vals * ww)
                return carry2
            lax.fori_loop(0, _GRP // _UNR, _group, 0)

            @pl.when(c + 2 < _NECH)
            def _():
                _edge_cp(c + 2, b).start()
        return carry
    lax.fori_loop(0, _NECH // 2, _pair, 0)

    @pl.when(half == 0)
    def _():
        pltpu.sync_copy(acc_rows, out0_hbm.at[pl.ds(r0, _DT)])

    @pl.when(half == 1)
    def _():
        pltpu.sync_copy(acc_rows, out1_hbm.at[pl.ds(r0, _DT)])


def _scatter(p, e3):
    return pl.kernel(
        _scat_body,
        out_type=(jax.ShapeDtypeStruct((_H, _NP), jnp.float32),
                  jax.ShapeDtypeStruct((_H, _NP), jnp.float32)),
        mesh=_sc_mesh(),
        scratch_types=[
            pltpu.VMEM((_DT, _NP), jnp.float32),
            pltpu.VMEM((_DT, _NP), jnp.float32),
            pltpu.VMEM((2, 3, _ECH), jnp.int32),
            pltpu.SemaphoreType.DMA,
            pltpu.SemaphoreType.DMA,
        ],
        compiler_params=_SC_PARAMS,
    )(p, e3)


def _z0_body(w0t_ref, xt_ref, z_ref):
    z_ref[...] = jnp.dot(w0t_ref[...], xt_ref[...],
                         preferred_element_type=jnp.float32)


def _scale_body(degp_ref, z_ref, dinv_ref, p_ref):
    deg = jnp.sum(degp_ref[...], axis=0, keepdims=True)
    dinv = lax.rsqrt(1.0 + deg)
    dinv_ref[...] = dinv
    p_ref[...] = z_ref[...] * dinv


def _mid_body(acc0_ref, acc1_ref, dinv_ref, b_ref, g_ref, bt_ref, wt_ref,
              p_ref):
    dinv = dinv_ref[...]
    acc = acc0_ref[...] + acc1_ref[...]
    h = (acc * dinv + b_ref[...]) * _SCALE * g_ref[...] + bt_ref[...]
    h = jnp.maximum(h, 0.0)
    p_ref[...] = jnp.dot(wt_ref[...], h,
                         preferred_element_type=jnp.float32) * dinv


def _fin_body(acc0_ref, acc1_ref, dinv_ref, b_ref, g_ref, bt_ref, batch_ref,
              sums_ref, cnt_ref):
    dinv = dinv_ref[...]
    acc = acc0_ref[...] + acc1_ref[...]
    h = (acc * dinv + b_ref[...]) * _SCALE * g_ref[...] + bt_ref[...]
    h = jnp.maximum(h, 0.0)
    oh = (batch_ref[...] ==
          lax.broadcasted_iota(jnp.int32, (_NP, _B), 1)).astype(jnp.float32)
    sums_ref[...] = jnp.dot(h, oh, preferred_element_type=jnp.float32)
    cnt_ref[...] = jnp.sum(oh, axis=0, keepdims=True)


def kernel(x, edge_index, edge_weight, batch,
           W0, b0, g0, bt0, W1, b1, g1, bt1, W2, b2, g2, bt2):
    f32 = jnp.float32
    src = edge_index[0]
    dst = edge_index[1]
    ew = edge_weight

    xt = jnp.pad(x, ((0, _NP - _N), (0, 0))).T
    batch_col = jnp.pad(batch.astype(jnp.int32), (0, _NP - _N),
                        constant_values=_B).reshape(_NP, 1)
    bcol = lambda v: v.reshape(_H, 1)

    degp = _deg_partials(dst, ew)
    z0 = pl.pallas_call(
        _z0_body, out_shape=jax.ShapeDtypeStruct((_H, _NP), f32))(W0.T, xt)
    dinv, p0 = pl.pallas_call(
        _scale_body,
        out_shape=[jax.ShapeDtypeStruct((1, _NP), f32),
                   jax.ShapeDtypeStruct((_H, _NP), f32)])(degp, z0)

    e3 = jnp.stack([src, dst,
                    jax.lax.bitcast_convert_type(ew, jnp.int32)])

    a0, a1 = _scatter(p0, e3)
    p1 = pl.pallas_call(
        _mid_body, out_shape=jax.ShapeDtypeStruct((_H, _NP), f32))(
            a0, a1, dinv, bcol(b0), bcol(g0), bcol(bt0), W1.T)

    a0, a1 = _scatter(p1, e3)
    p2 = pl.pallas_call(
        _mid_body, out_shape=jax.ShapeDtypeStruct((_H, _NP), f32))(
            a0, a1, dinv, bcol(b1), bcol(g1), bcol(bt1), W2.T)

    a0, a1 = _scatter(p2, e3)
    sums64, cnt = pl.pallas_call(
        _fin_body,
        out_shape=[jax.ShapeDtypeStruct((_H, _B), f32),
                   jax.ShapeDtypeStruct((1, _B), f32)])(
            a0, a1, dinv, bcol(b2), bcol(g2), bcol(bt2), batch_col)

    sums = sums64.T
    counts = jnp.clip(cnt[0], 1.0)
    mean = sums / counts[:, None]
    return jnp.concatenate([mean, sums], axis=-1)


# speedup vs baseline: 2.6354x; 2.6354x over previous
import functools

import jax
import jax.numpy as jnp
from jax import lax
from jax.experimental import pallas as pl
from jax.experimental.pallas import tpu as pltpu
from jax.experimental.pallas import tpu_sc as plsc

_N = 10000
_NP = 10240
_E = 320000
_DIN = 128
_H = 64
_B = 8
_EPS = 1e-5
_SCALE = float((1.0 + _EPS) ** -0.5)

_NC = 2
_NS = 16
_NT = _NC * _NS
_EP = _E // _NT
_CH = 10000
_NCH = _E // _CH


def _sc_mesh():
    return plsc.VectorSubcoreMesh(
        core_axis_name="c", subcore_axis_name="s",
        num_cores=_NC, num_subcores=_NS)


_SC_PARAMS = pltpu.CompilerParams(needs_layout_passes=False)


def _wid():
    return lax.axis_index("s") * _NC + lax.axis_index("c")


def _deg_body(dst_hbm, w_hbm, out_hbm, dst_buf, w_buf, deg_loc):
    w = _wid()
    base = pl.multiple_of(w * _EP, 8)
    pltpu.sync_copy(dst_hbm.at[pl.ds(base, _EP)], dst_buf)
    pltpu.sync_copy(w_hbm.at[pl.ds(base, _EP)], w_buf)

    def _zero(i, carry):
        deg_loc[pl.ds(i * 16, 16)] = jnp.zeros((16,), jnp.float32)
        return carry
    lax.fori_loop(0, _NP // 16, _zero, 0)

    def _edge(i, carry):
        d = dst_buf[pl.ds(i * 16, 16)]
        ww = jnp.abs(w_buf[pl.ds(i * 16, 16)])
        plsc.addupdate_scatter(deg_loc, [d], ww)
        return carry
    lax.fori_loop(0, _EP // 16, _edge, 0)

    pltpu.sync_copy(deg_loc, out_hbm.at[w])


def _deg_partials(dst, ew):
    return pl.kernel(
        _deg_body,
        out_type=jax.ShapeDtypeStruct((_NT, _NP), jnp.float32),
        mesh=_sc_mesh(),
        scratch_types=[
            pltpu.VMEM((_EP,), jnp.int32),
            pltpu.VMEM((_EP,), jnp.float32),
            pltpu.VMEM((_NP,), jnp.float32),
        ],
        compiler_params=_SC_PARAMS,
    )(dst, ew)


_DT = 4
_EH = _E // 2
_ECH = 3200
_NECH = _EH // _ECH
_GRP = _ECH // 16
_UNR = 4


def _scat_body(p_hbm, e3_hbm, out0_hbm, out1_hbm,
               p_rows, acc_rows, ebuf, sem0, sem1):
    w = _wid()
    half = w & 1
    dgrp = w >> 1
    r0 = pl.multiple_of(dgrp * _DT, 4)
    pltpu.sync_copy(p_hbm.at[pl.ds(r0, _DT)], p_rows)

    sems = [sem0, sem1]

    def _edge_cp(c, b):
        base = pl.multiple_of(half * _EH + c * _ECH, 128)
        return pltpu.make_async_copy(
            e3_hbm.at[:, pl.ds(base, _ECH)], ebuf.at[b], sems[b])

    _edge_cp(0, 0).start()
    _edge_cp(1, 1).start()

    keep = jnp.where(half == 0, 1.0, 0.0)

    def _init(i, carry):
        for j in range(_DT):
            acc_rows[j, pl.ds(i * 16, 16)] = p_rows[j, pl.ds(i * 16, 16)] * keep
        return carry
    lax.fori_loop(0, _NP // 16, _init, 0)

    def _pair(k, carry):
        for b in range(2):
            c = k * 2 + b
            _edge_cp(c, b).wait()

            @plsc.parallel_loop(0, _GRP, 1, unroll=_UNR)
            def _group(g):
                o = g * 16
                s = ebuf[b, 0, pl.ds(o, 16)]
                d = ebuf[b, 1, pl.ds(o, 16)]
                ww = jnp.abs(plsc.bitcast(ebuf[b, 2, pl.ds(o, 16)],
                                          jnp.float32))
                for j in range(_DT):
                    rj = jnp.full((16,), j, jnp.int32)
                    vals = plsc.load_gather(p_rows, [rj, s])
                    plsc.addupdate_scatter(acc_rows, [rj, d], vals * ww)

            @pl.when(c + 2 < _NECH)
            def _():
                _edge_cp(c + 2, b).start()
        return carry
    lax.fori_loop(0, _NECH // 2, _pair, 0)

    @pl.when(half == 0)
    def _():
        pltpu.sync_copy(acc_rows, out0_hbm.at[pl.ds(r0, _DT)])

    @pl.when(half == 1)
    def _():
        pltpu.sync_copy(acc_rows, out1_hbm.at[pl.ds(r0, _DT)])


def _scatter(p, e3):
    return pl.kernel(
        _scat_body,
        out_type=(jax.ShapeDtypeStruct((_H, _NP), jnp.float32),
                  jax.ShapeDtypeStruct((_H, _NP), jnp.float32)),
        mesh=_sc_mesh(),
        scratch_types=[
            pltpu.VMEM((_DT, _NP), jnp.float32),
            pltpu.VMEM((_DT, _NP), jnp.float32),
            pltpu.VMEM((2, 3, _ECH), jnp.int32),
            pltpu.SemaphoreType.DMA,
            pltpu.SemaphoreType.DMA,
        ],
        compiler_params=_SC_PARAMS,
    )(p, e3)


def _z0_body(w0t_ref, xt_ref, z_ref):
    z_ref[...] = jnp.dot(w0t_ref[...], xt_ref[...],
                         preferred_element_type=jnp.float32)


def _scale_body(degp_ref, z_ref, dinv_ref, p_ref):
    deg = jnp.sum(degp_ref[...], axis=0, keepdims=True)
    dinv = lax.rsqrt(1.0 + deg)
    dinv_ref[...] = dinv
    p_ref[...] = z_ref[...] * dinv


def _mid_body(acc0_ref, acc1_ref, dinv_ref, b_ref, g_ref, bt_ref, wt_ref,
              p_ref):
    dinv = dinv_ref[...]
    acc = acc0_ref[...] + acc1_ref[...]
    h = (acc * dinv + b_ref[...]) * _SCALE * g_ref[...] + bt_ref[...]
    h = jnp.maximum(h, 0.0)
    p_ref[...] = jnp.dot(wt_ref[...], h,
                         preferred_element_type=jnp.float32) * dinv


def _fin_body(acc0_ref, acc1_ref, dinv_ref, b_ref, g_ref, bt_ref, batch_ref,
              sums_ref, cnt_ref):
    dinv = dinv_ref[...]
    acc = acc0_ref[...] + acc1_ref[...]
    h = (acc * dinv + b_ref[...]) * _SCALE * g_ref[...] + bt_ref[...]
    h = jnp.maximum(h, 0.0)
    oh = (batch_ref[...] ==
          lax.broadcasted_iota(jnp.int32, (_NP, _B), 1)).astype(jnp.float32)
    sums_ref[...] = jnp.dot(h, oh, preferred_element_type=jnp.float32)
    cnt_ref[...] = jnp.sum(oh, axis=0, keepdims=True)


def kernel(x, edge_index, edge_weight, batch,
           W0, b0, g0, bt0, W1, b1, g1, bt1, W2, b2, g2, bt2):
    f32 = jnp.float32
    src = edge_index[0]
    dst = edge_index[1]
    ew = edge_weight

    xt = jnp.pad(x, ((0, _NP - _N), (0, 0))).T
    batch_col = jnp.pad(batch.astype(jnp.int32), (0, _NP - _N),
                        constant_values=_B).reshape(_NP, 1)
    bcol = lambda v: v.reshape(_H, 1)

    degp = _deg_partials(dst, ew)
    z0 = pl.pallas_call(
        _z0_body, out_shape=jax.ShapeDtypeStruct((_H, _NP), f32))(W0.T, xt)
    dinv, p0 = pl.pallas_call(
        _scale_body,
        out_shape=[jax.ShapeDtypeStruct((1, _NP), f32),
                   jax.ShapeDtypeStruct((_H, _NP), f32)])(degp, z0)

    e3 = jnp.stack([src, dst,
                    jax.lax.bitcast_convert_type(ew, jnp.int32)])

    a0, a1 = _scatter(p0, e3)
    p1 = pl.pallas_call(
        _mid_body, out_shape=jax.ShapeDtypeStruct((_H, _NP), f32))(
            a0, a1, dinv, bcol(b0), bcol(g0), bcol(bt0), W1.T)

    a0, a1 = _scatter(p1, e3)
    p2 = pl.pallas_call(
        _mid_body, out_shape=jax.ShapeDtypeStruct((_H, _NP), f32))(
            a0, a1, dinv, bcol(b1), bcol(g1), bcol(bt1), W2.T)

    a0, a1 = _scatter(p2, e3)
    sums64, cnt = pl.pallas_call(
        _fin_body,
        out_shape=[jax.ShapeDtypeStruct((_H, _B), f32),
                   jax.ShapeDtypeStruct((1, _B), f32)])(
            a0, a1, dinv, bcol(b2), bcol(g2), bcol(bt2), batch_col)

    sums = sums64.T
    counts = jnp.clip(cnt[0], 1.0)
    mean = sums / counts[:, None]
    return jnp.concatenate([mean, sums], axis=-1)
